# baseline (speedup 1.0000x reference)
import functools

import numpy as np
import jax
import jax.numpy as jnp
from jax import lax
from jax.experimental import pallas as pl
from jax.experimental.pallas import tpu as pltpu
from jax.experimental.pallas import tpu_sc as plsc

VOCAB = 1000000
D = 64
B = 1024
S = 200

NC = 2
NS = 16
NW = NC * NS
TOTAL = B * S
PER_W = TOTAL // NW
CHUNK = 160
NCHUNK = PER_W // CHUNK
NPAIR = NCHUNK // 2
SLICES = [(off, min(128, CHUNK - off)) for off in range(0, CHUNK, 128)]
NCOMB = 3 * S
SPLIT = 499968
SIZE0 = SPLIT
SIZE1 = VOCAB - SPLIT
BIG = 1.0e6


def _make_pe():
    pos = np.arange(S, dtype=np.float32)[:, None]
    div = np.exp(np.arange(0, D, 2, dtype=np.float32) * -(np.log(10000.0) / D))
    pe = np.zeros((S, D), dtype=np.float32)
    pe[:, 0::2] = np.sin(pos * div)
    pe[:, 1::2] = np.cos(pos * div)
    return pe


_PE = _make_pe()


def _comb_body(pe_ref, seg_ref, out_ref):
    pe = pe_ref[...]
    for l in range(3):
        x = pe + seg_ref[l, :][None, :]
        for p in range(2):
            col = (l * 2 + p) * 2 * D
            out_ref[:, col:col + D] = x - BIG * p
            out_ref[:, col + D:col + 2 * D] = x - BIG * (1 - p)


def _build_comb(seg_table):
    comb12 = pl.pallas_call(
        _comb_body,
        out_shape=jax.ShapeDtypeStruct((S, 12 * D), jnp.float32),
    )(_PE, seg_table)
    return comb12.reshape(2 * NCOMB, 2 * D)


_mesh = plsc.VectorSubcoreMesh(core_axis_name="c", subcore_axis_name="s")


@functools.partial(
    pl.kernel,
    mesh=_mesh,
    out_type=jax.ShapeDtypeStruct((TOTAL, D), jnp.float32),
    scratch_types=[
        pltpu.VMEM((CHUNK,), jnp.int32),
        pltpu.VMEM((CHUNK,), jnp.int32),
        pltpu.VMEM((CHUNK,), jnp.int32),
        pltpu.VMEM((CHUNK, D), jnp.float32),
        pltpu.VMEM((CHUNK, D), jnp.float32),
        pltpu.VMEM((CHUNK, 2 * D), jnp.float32),
        pltpu.VMEM((CHUNK, D), jnp.float32),
        pltpu.VMEM((CHUNK,), jnp.int32),
        pltpu.VMEM((CHUNK,), jnp.int32),
        pltpu.VMEM((CHUNK,), jnp.int32),
        pltpu.VMEM((CHUNK, D), jnp.float32),
        pltpu.VMEM((CHUNK, D), jnp.float32),
        pltpu.VMEM((CHUNK, 2 * D), jnp.float32),
        pltpu.VMEM((CHUNK, D), jnp.float32),
        pltpu.SemaphoreType.DMA,
        pltpu.SemaphoreType.DMA,
        pltpu.SemaphoreType.DMA,
        pltpu.SemaphoreType.DMA,
    ],
    compiler_params=pltpu.CompilerParams(use_tc_tiling_on_sc=False),
)
def _emb(seq_hbm, lab_hbm, tok0_hbm, tok1_hbm, comb_hbm, out_hbm,
         idx0_a, idx1_a, cmb_idx_a, tok0_va, tok1_va, cmb_va, out_va,
         idx0_b, idx1_b, cmb_idx_b, tok0_vb, tok1_vb, cmb_vb, out_vb,
         sem_ga, sem_gb, sem_oa, sem_ob):
    wid = lax.axis_index("s") * NC + lax.axis_index("c")
    w0 = wid * PER_W

    def prep_and_fire(c, idx0, idx1, cmb_idx, tok0_v, tok1_v, cmb_v, sem_g):
        base = pl.multiple_of(w0 + c * CHUNK, CHUNK)
        pltpu.sync_copy(seq_hbm.at[pl.ds(base, CHUNK)], idx0)
        pltpu.sync_copy(lab_hbm.at[pl.ds(base, CHUNK)], cmb_idx)
        for k in range(CHUNK // 16):
            sl = pl.ds(k * 16, 16)
            t = idx0[sl]
            bit = jnp.where(t >= SPLIT, 1, 0)
            idx0[sl] = jnp.minimum(t, SIZE0 - 1)
            idx1[sl] = jnp.maximum(t - SPLIT, 0)
            v = base + k * 16 + lax.iota(jnp.int32, 16)
            cmb_idx[sl] = (lax.rem(v, S) * 3 + cmb_idx[sl]) * 2 + bit
        for off, ln in SLICES:
            pltpu.async_copy(tok0_hbm.at[idx0.at[pl.ds(off, ln)]],
                             tok0_v.at[pl.ds(off, ln)], sem_g)
            pltpu.async_copy(tok1_hbm.at[idx1.at[pl.ds(off, ln)]],
                             tok1_v.at[pl.ds(off, ln)], sem_g)
            pltpu.async_copy(comb_hbm.at[cmb_idx.at[pl.ds(off, ln)]],
                             cmb_v.at[pl.ds(off, ln)], sem_g)

    def wait_gathers(tok0_v, tok1_v, cmb_v, sem_g):
        pltpu.make_async_copy(tok0_hbm.at[pl.ds(0, CHUNK)], tok0_v, sem_g).wait()
        pltpu.make_async_copy(tok0_hbm.at[pl.ds(0, CHUNK)], tok1_v, sem_g).wait()
        pltpu.make_async_copy(comb_hbm.at[pl.ds(0, CHUNK)], cmb_v, sem_g).wait()

    def add_chunk(tok0_v, tok1_v, cmb_v, out_v):
        def add_rows(r, carry2):
            for j in range(2):
                i = r * 2 + j
                for k in range(D // 16):
                    sl = pl.ds(k * 16, 16)
                    a = tok0_v[i, sl] + cmb_v[i, sl]
                    b = tok1_v[i, sl] + cmb_v[i, pl.ds(D + k * 16, 16)]
                    out_v[i, sl] = jnp.maximum(a, b)
            return carry2

        lax.fori_loop(0, CHUNK // 2, add_rows, 0)

    def fire_writeback(c, out_v, sem_o):
        base = pl.multiple_of(w0 + c * CHUNK, CHUNK)
        pltpu.async_copy(out_v, out_hbm.at[pl.ds(base, CHUNK)], sem_o)

    def wait_writeback(out_v, sem_o):
        pltpu.make_async_copy(out_v, out_hbm.at[pl.ds(0, CHUNK)], sem_o).wait()

    prep_and_fire(0, idx0_a, idx1_a, cmb_idx_a, tok0_va, tok1_va, cmb_va,
                  sem_ga)
    prep_and_fire(1, idx0_b, idx1_b, cmb_idx_b, tok0_vb, tok1_vb, cmb_vb,
                  sem_gb)

    def pair(i, carry):
        ca = 2 * i
        cb = 2 * i + 1

        @pl.when(i > 0)
        def _():
            wait_writeback(out_va, sem_oa)

        wait_gathers(tok0_va, tok1_va, cmb_va, sem_ga)
        add_chunk(tok0_va, tok1_va, cmb_va, out_va)
        fire_writeback(ca, out_va, sem_oa)

        @pl.when(i < NPAIR - 1)
        def _():
            prep_and_fire(ca + 2, idx0_a, idx1_a, cmb_idx_a,
                          tok0_va, tok1_va, cmb_va, sem_ga)

        @pl.when(i > 0)
        def _():
            wait_writeback(out_vb, sem_ob)

        wait_gathers(tok0_vb, tok1_vb, cmb_vb, sem_gb)
        add_chunk(tok0_vb, tok1_vb, cmb_vb, out_vb)
        fire_writeback(cb, out_vb, sem_ob)

        @pl.when(i < NPAIR - 1)
        def _():
            prep_and_fire(cb + 2, idx0_b, idx1_b, cmb_idx_b,
                          tok0_vb, tok1_vb, cmb_vb, sem_gb)

        return carry

    lax.fori_loop(0, NPAIR, pair, 0)
    wait_writeback(out_va, sem_oa)
    wait_writeback(out_vb, sem_ob)


def kernel(sequence, segment_label, tok_table, seg_table):
    comb = _build_comb(seg_table)
    seq = sequence.reshape(TOTAL)
    lab = segment_label.reshape(TOTAL)
    tok0 = lax.slice(tok_table, (0, 0), (SPLIT, D))
    tok1 = lax.slice(tok_table, (SPLIT, 0), (VOCAB, D))
    out = _emb(seq, lab, tok0, tok1, comb)
    return out.reshape(B, S, D)
